# baseline (speedup 1.0000x reference)
import functools

import jax
import jax.numpy as jnp
from jax import lax
from jax.experimental import pallas as pl
from jax.experimental.pallas import tpu as pltpu
from jax.experimental.pallas import tpu_sc as plsc

_BETA = 0.25
_B, _C, _L = 8, 32, 1024
_K = 8192
_T = _B * _L
_TT = 1024
_KT = 2048
_NTOK = float(_B * _C * _L)


def _vq_argmax_body(xt_ref, cb_ref, iota_ref, code_ref, loss_ref,
                    bestv_ref, besti_ref, a2_ref, b2_ref):
    t = pl.program_id(0)
    k = pl.program_id(1)
    nk = pl.num_programs(1)

    x = xt_ref[...]
    cb = cb_ref[...]
    ab = lax.dot_general(
        x, cb, (((1,), (1,)), ((), ())),
        preferred_element_type=jnp.float32)

    @pl.when(k == 0)
    def _():
        a2_ref[...] = jnp.sum(x * x, axis=1, keepdims=True)

    @pl.when(t == 0)
    def _():
        b2_ref[:, pl.ds(k * _KT, _KT)] = jnp.sum(cb * cb, axis=1)[None, :]

    a2 = a2_ref[...]
    b2 = b2_ref[:, pl.ds(k * _KT, _KT)]
    d2 = (a2 + b2) - 2.0 * ab

    mu = jnp.max(d2, axis=1, keepdims=True)
    gidx = iota_ref[...]
    cand = jnp.where(d2 == mu, gidx, jnp.float32(_K))
    idxf = jnp.min(cand, axis=1, keepdims=True)
    m = jnp.maximum(mu, 0.0)
    kbase = (k * _KT).astype(jnp.float32)
    idx = jnp.where(mu > 0.0, idxf, kbase).astype(jnp.int32)

    @pl.when(k == 0)
    def _():
        bestv_ref[...] = m
        besti_ref[...] = idx

    @pl.when(k > 0)
    def _():
        better = m > bestv_ref[...]
        bestv_ref[...] = jnp.where(better, m, bestv_ref[...])
        besti_ref[...] = jnp.where(better, idx, besti_ref[...])

    code_ref[...] = besti_ref[...]

    @pl.when((t == 0) & (k == 0))
    def _():
        loss_ref[0, 0] = 0.0

    @pl.when(k == nk - 1)
    def _():
        loss_ref[0, 0] += jnp.sum(bestv_ref[...])

    @pl.when((t == pl.num_programs(0) - 1) & (k == nk - 1))
    def _():
        loss_ref[0, 0] = loss_ref[0, 0] * (1.0 / _NTOK)


def _argmax_and_loss(xt, codebook):
    grid = (_T // _TT, _K // _KT)
    iota_f = jnp.arange(_K, dtype=jnp.float32)[None, :]
    return pl.pallas_call(
        _vq_argmax_body,
        grid=grid,
        in_specs=[
            pl.BlockSpec((_TT, _C), lambda t, k: (t, 0)),
            pl.BlockSpec((_KT, _C), lambda t, k: (k, 0)),
            pl.BlockSpec((1, _KT), lambda t, k: (0, k)),
        ],
        out_specs=[
            pl.BlockSpec((_TT, 1), lambda t, k: (t, 0)),
            pl.BlockSpec(memory_space=pltpu.SMEM),
        ],
        out_shape=[
            jax.ShapeDtypeStruct((_T, 1), jnp.int32),
            jax.ShapeDtypeStruct((1, 1), jnp.float32),
        ],
        scratch_shapes=[
            pltpu.VMEM((_TT, 1), jnp.float32),
            pltpu.VMEM((_TT, 1), jnp.int32),
            pltpu.VMEM((_TT, 1), jnp.float32),
            pltpu.VMEM((1, _K), jnp.float32),
        ],
    )(xt, codebook, iota_f)


_GATHER_CHUNK = 128
_CPAD = 128


def _sc_gather_body(cb_hbm, idx_hbm, out_hbm, idx_v, rows_v, sem):
    info = plsc.get_sparse_core_info()
    nc, ns = info.num_cores, info.num_subcores
    bpw = _T // (nc * ns)
    wid = lax.axis_index("s") * nc + lax.axis_index("c")
    base = wid * bpw
    pltpu.sync_copy(idx_hbm.at[pl.ds(base, bpw)], idx_v)
    copies = []
    for j in range(bpw // _GATHER_CHUNK):
        sl = pl.ds(j * _GATHER_CHUNK, _GATHER_CHUNK)
        copies.append(
            pltpu.async_copy(cb_hbm.at[idx_v.at[sl]], rows_v.at[sl], sem))
    for c in copies:
        c.wait()
    pltpu.sync_copy(rows_v, out_hbm.at[pl.ds(base, bpw)])


def _sc_gather(cb_pad, code_flat):
    info = plsc.get_sparse_core_info()
    bpw = _T // (info.num_cores * info.num_subcores)
    mesh = plsc.VectorSubcoreMesh(core_axis_name="c", subcore_axis_name="s")
    return pl.kernel(
        _sc_gather_body,
        out_type=jax.ShapeDtypeStruct((_T, _CPAD), jnp.float32),
        mesh=mesh,
        scratch_types=[
            pltpu.VMEM((bpw,), jnp.int32),
            pltpu.VMEM((bpw, _CPAD), jnp.float32),
            pltpu.SemaphoreType.DMA,
        ],
    )(cb_pad, code_flat)


def kernel(x, codebook):
    xt = jnp.transpose(x, (0, 2, 1)).reshape(_T, _C)
    code2d, loss2d = _argmax_and_loss(xt, codebook)
    code_flat = code2d.reshape(_T)
    cb_pad = jnp.pad(codebook, ((0, 0), (0, _CPAD - _C)))
    rows = _sc_gather(cb_pad, code_flat)[:, :_C]
    emb = jnp.transpose(rows.reshape(_B, _L, _C), (0, 2, 1))
    code = code_flat.reshape(_B, _L)
    loss = loss2d.reshape(())
    return (code, emb, loss)


# speedup vs baseline: 1.1914x; 1.1914x over previous
import functools

import jax
import jax.numpy as jnp
from jax import lax
from jax.experimental import pallas as pl
from jax.experimental.pallas import tpu as pltpu
from jax.experimental.pallas import tpu_sc as plsc

_BETA = 0.25
_B, _C, _L = 8, 32, 1024
_K = 8192
_T = _B * _L
_TT = 1024
_KT = 2048
_NTOK = float(_B * _C * _L)


def _vq_argmax_body(xt_ref, cb_ref, code_ref, loss_ref,
                    bestv_ref, besti_ref):
    t = pl.program_id(0)
    k = pl.program_id(1)
    nk = pl.num_programs(1)

    x = xt_ref[...]
    cb = cb_ref[...]
    ab = lax.dot_general(
        x, cb, (((1,), (1,)), ((), ())),
        preferred_element_type=jnp.float32)

    a2 = jnp.sum(x * x, axis=1, keepdims=True)
    b2 = jnp.sum(cb * cb, axis=1)[None, :]
    d2 = (a2 + b2) - 2.0 * ab

    mu = jnp.max(d2, axis=1, keepdims=True)
    gidx = (lax.broadcasted_iota(jnp.int32, d2.shape, 1)
            + k * _KT).astype(jnp.float32)
    cand = jnp.where(d2 == mu, gidx, jnp.float32(_K))
    idxf = jnp.min(cand, axis=1, keepdims=True)
    m = jnp.maximum(mu, 0.0)
    kbase = (k * _KT).astype(jnp.float32)
    idx = jnp.where(mu > 0.0, idxf, kbase).astype(jnp.int32)

    @pl.when(k == 0)
    def _():
        bestv_ref[...] = m
        besti_ref[...] = idx

    @pl.when(k > 0)
    def _():
        better = m > bestv_ref[...]
        bestv_ref[...] = jnp.where(better, m, bestv_ref[...])
        besti_ref[...] = jnp.where(better, idx, besti_ref[...])

    code_ref[...] = besti_ref[...]

    @pl.when((t == 0) & (k == 0))
    def _():
        loss_ref[0, 0] = 0.0

    @pl.when(k == nk - 1)
    def _():
        loss_ref[0, 0] += jnp.sum(bestv_ref[...])

    @pl.when((t == pl.num_programs(0) - 1) & (k == nk - 1))
    def _():
        loss_ref[0, 0] = loss_ref[0, 0] * (1.0 / _NTOK)


def _argmax_and_loss(xt, codebook):
    grid = (_T // _TT, _K // _KT)
    return pl.pallas_call(
        _vq_argmax_body,
        grid=grid,
        in_specs=[
            pl.BlockSpec((_TT, _C), lambda t, k: (t, 0)),
            pl.BlockSpec((_KT, _C), lambda t, k: (k, 0)),
        ],
        out_specs=[
            pl.BlockSpec((_TT, 1), lambda t, k: (t, 0)),
            pl.BlockSpec(memory_space=pltpu.SMEM),
        ],
        out_shape=[
            jax.ShapeDtypeStruct((_T, 1), jnp.int32),
            jax.ShapeDtypeStruct((1, 1), jnp.float32),
        ],
        scratch_shapes=[
            pltpu.VMEM((_TT, 1), jnp.float32),
            pltpu.VMEM((_TT, 1), jnp.int32),
        ],
    )(xt, codebook)


_GATHER_CHUNK = 128
_CPAD = 128


def _sc_gather_body(cb_hbm, idx_hbm, out_hbm, idx_v, rows_v, sem):
    info = plsc.get_sparse_core_info()
    nc, ns = info.num_cores, info.num_subcores
    bpw = _T // (nc * ns)
    wid = lax.axis_index("s") * nc + lax.axis_index("c")
    base = wid * bpw
    pltpu.sync_copy(idx_hbm.at[pl.ds(base, bpw)], idx_v)
    copies = []
    for j in range(bpw // _GATHER_CHUNK):
        sl = pl.ds(j * _GATHER_CHUNK, _GATHER_CHUNK)
        copies.append(
            pltpu.async_copy(cb_hbm.at[idx_v.at[sl]], rows_v.at[sl], sem))
    for c in copies:
        c.wait()
    pltpu.sync_copy(rows_v, out_hbm.at[pl.ds(base, bpw)])


def _sc_gather(cb_pad, code_flat):
    info = plsc.get_sparse_core_info()
    bpw = _T // (info.num_cores * info.num_subcores)
    mesh = plsc.VectorSubcoreMesh(core_axis_name="c", subcore_axis_name="s")
    return pl.kernel(
        _sc_gather_body,
        out_type=jax.ShapeDtypeStruct((_T, _CPAD), jnp.float32),
        mesh=mesh,
        scratch_types=[
            pltpu.VMEM((bpw,), jnp.int32),
            pltpu.VMEM((bpw, _CPAD), jnp.float32),
            pltpu.SemaphoreType.DMA,
        ],
    )(cb_pad, code_flat)


def kernel(x, codebook):
    xt = jnp.transpose(x, (0, 2, 1)).reshape(_T, _C)
    code2d, loss2d = _argmax_and_loss(xt, codebook)
    code_flat = code2d.reshape(_T)
    cb_pad = jnp.pad(codebook, ((0, 0), (0, _CPAD - _C)))
    rows = _sc_gather(cb_pad, code_flat)[:, :_C]
    emb = jnp.transpose(rows.reshape(_B, _L, _C), (0, 2, 1))
    code = code_flat.reshape(_B, _L)
    loss = loss2d.reshape(())
    return (code, emb, loss)


# speedup vs baseline: 1.2764x; 1.0713x over previous
import functools

import jax
import jax.numpy as jnp
from jax import lax
from jax.experimental import pallas as pl
from jax.experimental.pallas import tpu as pltpu
from jax.experimental.pallas import tpu_sc as plsc

_BETA = 0.25
_B, _C, _L = 8, 32, 1024
_K = 8192
_T = _B * _L
_TT = 1024
_KT = 2048
_NTOK = float(_B * _C * _L)


def _vq_argmax_body(x_ref, cb_ref, code_ref, loss_ref,
                    bestv_ref, besti_ref):
    t = pl.program_id(0)
    k = pl.program_id(1)
    nk = pl.num_programs(1)

    xb = x_ref[0]
    cb = cb_ref[...]
    ab = lax.dot_general(
        cb, xb, (((1,), (0,)), ((), ())),
        preferred_element_type=jnp.float32)

    a2 = jnp.sum(xb * xb, axis=0, keepdims=True)
    b2 = jnp.sum(cb * cb, axis=1, keepdims=True)
    d2 = (a2 + b2) - 2.0 * ab

    mu = jnp.max(d2, axis=0, keepdims=True)
    gidx = (lax.broadcasted_iota(jnp.int32, d2.shape, 0)
            + k * _KT).astype(jnp.float32)
    cand = jnp.where(d2 == mu, gidx, jnp.float32(_K))
    idxf = jnp.min(cand, axis=0, keepdims=True)
    m = jnp.maximum(mu, 0.0)
    kbase = (k * _KT).astype(jnp.float32)
    idx = jnp.where(mu > 0.0, idxf, kbase).astype(jnp.int32)

    @pl.when(k == 0)
    def _():
        bestv_ref[...] = m
        besti_ref[...] = idx

    @pl.when(k > 0)
    def _():
        better = m > bestv_ref[...]
        bestv_ref[...] = jnp.where(better, m, bestv_ref[...])
        besti_ref[...] = jnp.where(better, idx, besti_ref[...])

    code_ref[0] = besti_ref[...]

    @pl.when((t == 0) & (k == 0))
    def _():
        loss_ref[0, 0] = 0.0

    @pl.when(k == nk - 1)
    def _():
        loss_ref[0, 0] += jnp.sum(bestv_ref[...])

    @pl.when((t == pl.num_programs(0) - 1) & (k == nk - 1))
    def _():
        loss_ref[0, 0] = loss_ref[0, 0] * (1.0 / _NTOK)


def _argmax_and_loss(x, codebook):
    grid = (_B, _K // _KT)
    return pl.pallas_call(
        _vq_argmax_body,
        grid=grid,
        in_specs=[
            pl.BlockSpec((1, _C, _L), lambda t, k: (t, 0, 0)),
            pl.BlockSpec((_KT, _C), lambda t, k: (k, 0)),
        ],
        out_specs=[
            pl.BlockSpec((1, 1, _L), lambda t, k: (t, 0, 0)),
            pl.BlockSpec(memory_space=pltpu.SMEM),
        ],
        out_shape=[
            jax.ShapeDtypeStruct((_B, 1, _L), jnp.int32),
            jax.ShapeDtypeStruct((1, 1), jnp.float32),
        ],
        scratch_shapes=[
            pltpu.VMEM((1, _L), jnp.float32),
            pltpu.VMEM((1, _L), jnp.int32),
        ],
    )(x, codebook)


_GATHER_CHUNK = 128
_CPAD = 128


def _sc_gather_body(cb_hbm, idx_hbm, out_hbm, idx_v, rows_v, sem):
    info = plsc.get_sparse_core_info()
    nc, ns = info.num_cores, info.num_subcores
    bpw = _T // (nc * ns)
    wid = lax.axis_index("s") * nc + lax.axis_index("c")
    base = wid * bpw
    pltpu.sync_copy(idx_hbm.at[pl.ds(base, bpw)], idx_v)
    copies = []
    for j in range(bpw // _GATHER_CHUNK):
        sl = pl.ds(j * _GATHER_CHUNK, _GATHER_CHUNK)
        copies.append(
            pltpu.async_copy(cb_hbm.at[idx_v.at[sl]], rows_v.at[sl], sem))
    for c in copies:
        c.wait()
    pltpu.sync_copy(rows_v, out_hbm.at[pl.ds(base, bpw)])


def _sc_gather(cb_pad, code_flat):
    info = plsc.get_sparse_core_info()
    bpw = _T // (info.num_cores * info.num_subcores)
    mesh = plsc.VectorSubcoreMesh(core_axis_name="c", subcore_axis_name="s")
    return pl.kernel(
        _sc_gather_body,
        out_type=jax.ShapeDtypeStruct((_T, _CPAD), jnp.float32),
        mesh=mesh,
        scratch_types=[
            pltpu.VMEM((bpw,), jnp.int32),
            pltpu.VMEM((bpw, _CPAD), jnp.float32),
            pltpu.SemaphoreType.DMA,
        ],
    )(cb_pad, code_flat)


def kernel(x, codebook):
    code3d, loss2d = _argmax_and_loss(x, codebook)
    code_flat = code3d.reshape(_T)
    cb_pad = jnp.pad(codebook, ((0, 0), (0, _CPAD - _C)))
    rows = _sc_gather(cb_pad, code_flat)[:, :_C]
    emb = jnp.transpose(rows.reshape(_B, _L, _C), (0, 2, 1))
    code = code_flat.reshape(_B, _L)
    loss = loss2d.reshape(())
    return (code, emb, loss)


# speedup vs baseline: 1.3482x; 1.0563x over previous
import functools

import jax
import jax.numpy as jnp
from jax import lax
from jax.experimental import pallas as pl
from jax.experimental.pallas import tpu as pltpu
from jax.experimental.pallas import tpu_sc as plsc

_BETA = 0.25
_B, _C, _L = 8, 32, 1024
_K = 8192
_T = _B * _L
_TT = 1024
_KT = 4096
_NTOK = float(_B * _C * _L)


def _vq_argmax_body(x_ref, cb_ref, code_ref, loss_ref):
    t = pl.program_id(0)

    xb = x_ref[0]
    cb = cb_ref[...]
    ab = lax.dot_general(
        cb, xb, (((1,), (0,)), ((), ())),
        preferred_element_type=jnp.float32)

    a2 = jnp.sum(xb * xb, axis=0, keepdims=True)
    b2 = jnp.sum(cb * cb, axis=1, keepdims=True)
    d2 = (a2 + b2) - 2.0 * ab

    mu = jnp.max(d2, axis=0, keepdims=True)
    gidx = lax.broadcasted_iota(jnp.int32, d2.shape, 0).astype(jnp.float32)
    cand = jnp.where(d2 == mu, gidx, jnp.float32(_K))
    idxf = jnp.min(cand, axis=0, keepdims=True)
    m = jnp.maximum(mu, 0.0)
    idx = jnp.where(mu > 0.0, idxf, 0.0).astype(jnp.int32)

    code_ref[0] = idx

    @pl.when(t == 0)
    def _():
        loss_ref[0, 0] = 0.0

    loss_ref[0, 0] += jnp.sum(m)

    @pl.when(t == pl.num_programs(0) - 1)
    def _():
        loss_ref[0, 0] = loss_ref[0, 0] * (1.0 / _NTOK)


def _argmax_and_loss(x, codebook):
    grid = (_B,)
    return pl.pallas_call(
        _vq_argmax_body,
        grid=grid,
        in_specs=[
            pl.BlockSpec((1, _C, _L), lambda t: (t, 0, 0)),
            pl.BlockSpec((_K, _C), lambda t: (0, 0)),
        ],
        out_specs=[
            pl.BlockSpec((1, 1, _L), lambda t: (t, 0, 0)),
            pl.BlockSpec(memory_space=pltpu.SMEM),
        ],
        out_shape=[
            jax.ShapeDtypeStruct((_B, 1, _L), jnp.int32),
            jax.ShapeDtypeStruct((1, 1), jnp.float32),
        ],
    )(x, codebook)


_GATHER_CHUNK = 128
_CPAD = 128


def _sc_gather_body(cb_hbm, idx_hbm, out_hbm, idx_v, rows_v, sem):
    info = plsc.get_sparse_core_info()
    nc, ns = info.num_cores, info.num_subcores
    bpw = _T // (nc * ns)
    wid = lax.axis_index("s") * nc + lax.axis_index("c")
    base = wid * bpw
    pltpu.sync_copy(idx_hbm.at[pl.ds(base, bpw)], idx_v)
    copies = []
    for j in range(bpw // _GATHER_CHUNK):
        sl = pl.ds(j * _GATHER_CHUNK, _GATHER_CHUNK)
        copies.append(
            pltpu.async_copy(cb_hbm.at[idx_v.at[sl]], rows_v.at[sl], sem))
    for c in copies:
        c.wait()
    pltpu.sync_copy(rows_v, out_hbm.at[pl.ds(base, bpw)])


def _sc_gather(cb_pad, code_flat):
    info = plsc.get_sparse_core_info()
    bpw = _T // (info.num_cores * info.num_subcores)
    mesh = plsc.VectorSubcoreMesh(core_axis_name="c", subcore_axis_name="s")
    return pl.kernel(
        _sc_gather_body,
        out_type=jax.ShapeDtypeStruct((_T, _CPAD), jnp.float32),
        mesh=mesh,
        scratch_types=[
            pltpu.VMEM((bpw,), jnp.int32),
            pltpu.VMEM((bpw, _CPAD), jnp.float32),
            pltpu.SemaphoreType.DMA,
        ],
    )(cb_pad, code_flat)


def kernel(x, codebook):
    code3d, loss2d = _argmax_and_loss(x, codebook)
    code_flat = code3d.reshape(_T)
    cb_pad = jnp.pad(codebook, ((0, 0), (0, _CPAD - _C)))
    rows = _sc_gather(cb_pad, code_flat)[:, :_C]
    emb = jnp.transpose(rows.reshape(_B, _L, _C), (0, 2, 1))
    code = code_flat.reshape(_B, _L)
    loss = loss2d.reshape(())
    return (code, emb, loss)
